# baseline (speedup 1.0000x reference)
import functools

import jax
import jax.numpy as jnp
from jax import lax
from jax.experimental import pallas as pl
from jax.experimental.pallas import tpu as pltpu
from jax.experimental.pallas import tpu_sc as plsc

B = 32
C = 2
N = 8192
D = 485
MIN_D = 44
MAX_LFO = 441
PAD = 496
L = 16
NG = N // L


def _flanger_body(x_hbm, mod_hbm, fb_hbm, w_hbm, dp_hbm, mx_hbm, out_hbm,
                  modv, x0, x1, h0, h1, o0, o1, pvec, sem_a, sem_b, sem_o):
    b = lax.axis_index("s") * 2 + lax.axis_index("c")

    cps_a = [
        pltpu.async_copy(mod_hbm.at[b], modv.at[pl.ds(0, N)], sem_a),
        pltpu.async_copy(fb_hbm, pvec.at[pl.ds(0, B)], sem_a),
        pltpu.async_copy(w_hbm, pvec.at[pl.ds(B, B)], sem_a),
        pltpu.async_copy(dp_hbm, pvec.at[pl.ds(2 * B, B)], sem_a),
        pltpu.async_copy(mx_hbm, pvec.at[pl.ds(3 * B, B)], sem_a),
    ]
    cps_b = [
        pltpu.async_copy(x_hbm.at[b, 0], x0, sem_b),
        pltpu.async_copy(x_hbm.at[b, 1], x1, sem_b),
    ]

    zero = jnp.zeros((L,), jnp.float32)
    for j in range(PAD // L):
        h0[pl.ds(j * L, L)] = zero
        h1[pl.ds(j * L, L)] = zero

    for cp in cps_a:
        cp.wait()
    modv[pl.ds(N, L)] = zero
    modv[pl.ds(N + L, L)] = zero

    bvec = jnp.full((L,), b, jnp.int32)
    fb = plsc.load_gather(pvec, [bvec])
    wd = plsc.load_gather(pvec, [bvec + B])
    dp = plsc.load_gather(pvec, [bvec + 2 * B])
    mx = plsc.load_gather(pvec, [bvec + 3 * B])
    cw = jnp.float32(MAX_LFO) * wd
    mdp = mx * dp

    lanes = lax.iota(jnp.int32, L)
    fD = jnp.float32(D)
    fMIN = jnp.float32(MIN_D)

    def idxmath(off, wq0):
        tp = (off + PAD - D) + lanes
        wqv = wq0 + lanes
        wqv = jnp.where(wqv >= D, wqv - D, wqv)
        wf = wqv.astype(jnp.float32)
        mv = modv[pl.ds(pl.multiple_of(off, L), L)]
        delay = cw * mv + fMIN
        a = wf - delay + fD
        pa = a.astype(jnp.int32)
        fr = a - pa.astype(jnp.float32)
        wq1 = wq0 + L
        wq1 = jnp.where(wq1 >= D, wq1 - D, wq1)
        return tp - wqv + pa, fr, wq1

    def gather4(gi):
        gn = gi + 1
        return (plsc.load_gather(h0, [gi]), plsc.load_gather(h0, [gn]),
                plsc.load_gather(h1, [gi]), plsc.load_gather(h1, [gn]))

    def emit_group(off, frc, taps):
        pv0, nv0, pv1, nv1 = taps
        om = jnp.float32(1.0) - frc
        for (xr, hr, orr, pv, nv) in ((x0, h0, o0, pv0, nv0),
                                      (x1, h1, o1, pv1, nv1)):
            iv = frc * nv + om * pv
            xv = xr[pl.ds(pl.multiple_of(off, L), L)]
            hr[pl.ds(pl.multiple_of(off + PAD, L), L)] = xv + fb * iv
            orr[pl.ds(pl.multiple_of(off, L), L)] = xv + mdp * iv

    g0, f0, wq1 = idxmath(jnp.int32(0), jnp.int32(0))
    g1, f1, wq2 = idxmath(jnp.int32(L), wq1)
    taps0 = gather4(g0)

    for cp in cps_b:
        cp.wait()

    def step(jj, carry):
        fra, ta0, ta1, ta2, ta3, gb, frb, wqc = carry
        off_a = jj * (2 * L)
        taps_b = gather4(gb)
        emit_group(off_a, fra, (ta0, ta1, ta2, ta3))
        gc, frc, wqd = idxmath(off_a + 2 * L, wqc)
        taps_c = gather4(gc)
        emit_group(off_a + L, frb, taps_b)
        gd, frd, wqe = idxmath(off_a + 3 * L, wqd)
        return (frc, *taps_c, gd, frd, wqe)

    half = lax.fori_loop(0, NG // 4, step, (f0, *taps0, g1, f1, wq2))
    ocp = [pltpu.async_copy(o0.at[pl.ds(0, N // 2)],
                            out_hbm.at[b, 0, pl.ds(0, N // 2)], sem_o),
           pltpu.async_copy(o1.at[pl.ds(0, N // 2)],
                            out_hbm.at[b, 1, pl.ds(0, N // 2)], sem_o)]
    lax.fori_loop(NG // 4, NG // 2, step, half)

    pltpu.sync_copy(o0.at[pl.ds(N // 2, N // 2)],
                    out_hbm.at[b, 0, pl.ds(N // 2, N // 2)])
    pltpu.sync_copy(o1.at[pl.ds(N // 2, N // 2)],
                    out_hbm.at[b, 1, pl.ds(N // 2, N // 2)])
    for cp in ocp:
        cp.wait()


@jax.jit
def kernel(x, mod_sig, feedback, width, depth, mix):
    mesh = plsc.VectorSubcoreMesh(core_axis_name="c", subcore_axis_name="s")
    f = functools.partial(
        pl.kernel,
        mesh=mesh,
        compiler_params=pltpu.CompilerParams(needs_layout_passes=False),
        out_type=jax.ShapeDtypeStruct((B, C, N), jnp.float32),
        scratch_types=[
            pltpu.VMEM((N + 2 * L,), jnp.float32),
            pltpu.VMEM((N,), jnp.float32),
            pltpu.VMEM((N,), jnp.float32),
            pltpu.VMEM((PAD + N,), jnp.float32),
            pltpu.VMEM((PAD + N,), jnp.float32),
            pltpu.VMEM((N,), jnp.float32),
            pltpu.VMEM((N,), jnp.float32),
            pltpu.VMEM((4 * B,), jnp.float32),
            pltpu.SemaphoreType.DMA,
            pltpu.SemaphoreType.DMA,
            pltpu.SemaphoreType.DMA,
        ],
    )(_flanger_body)
    return f(x, mod_sig, feedback, width, depth, mix)


# speedup vs baseline: 1.0176x; 1.0176x over previous
import functools

import jax
import jax.numpy as jnp
from jax import lax
from jax.experimental import pallas as pl
from jax.experimental.pallas import tpu as pltpu
from jax.experimental.pallas import tpu_sc as plsc

B = 32
C = 2
N = 8192
D = 485
MIN_D = 44
MAX_LFO = 441
PAD = 496
L = 16
NG = N // L


def _flanger_body(x_hbm, mod_hbm, fb_hbm, w_hbm, dp_hbm, mx_hbm, out_hbm,
                  modv, x0, x1, h0, h1, o0, o1, pvec, sem_a, sem_b, sem_o):
    b = lax.axis_index("s") * 2 + lax.axis_index("c")

    cps_a = [
        pltpu.async_copy(mod_hbm.at[b], modv.at[pl.ds(0, N)], sem_a),
        pltpu.async_copy(fb_hbm, pvec.at[pl.ds(0, B)], sem_a),
        pltpu.async_copy(w_hbm, pvec.at[pl.ds(B, B)], sem_a),
        pltpu.async_copy(dp_hbm, pvec.at[pl.ds(2 * B, B)], sem_a),
        pltpu.async_copy(mx_hbm, pvec.at[pl.ds(3 * B, B)], sem_a),
    ]
    cps_b = [
        pltpu.async_copy(x_hbm.at[b, 0], x0, sem_b),
        pltpu.async_copy(x_hbm.at[b, 1], x1, sem_b),
    ]

    zero = jnp.zeros((L,), jnp.float32)
    for j in range(PAD // L):
        h0[pl.ds(j * L, L)] = zero
        h1[pl.ds(j * L, L)] = zero

    for cp in cps_a:
        cp.wait()
    modv[pl.ds(N, L)] = zero
    modv[pl.ds(N + L, L)] = zero

    bvec = jnp.full((L,), b, jnp.int32)
    fb = plsc.load_gather(pvec, [bvec])
    wd = plsc.load_gather(pvec, [bvec + B])
    dp = plsc.load_gather(pvec, [bvec + 2 * B])
    mx = plsc.load_gather(pvec, [bvec + 3 * B])
    cw = jnp.float32(MAX_LFO) * wd
    mdp = mx * dp

    lanes = lax.iota(jnp.int32, L)
    fD = jnp.float32(D)
    fMIN = jnp.float32(MIN_D)

    def idxmath(off, wq0):
        tp = (off + PAD - D) + lanes
        wqv = wq0 + lanes
        wqv = jnp.where(wqv >= D, wqv - D, wqv)
        wf = wqv.astype(jnp.float32)
        mv = modv[pl.ds(pl.multiple_of(off, L), L)]
        delay = cw * mv + fMIN
        a = wf - delay + fD
        pa = a.astype(jnp.int32)
        fr = a - pa.astype(jnp.float32)
        om = jnp.float32(1.0) - fr
        wq1 = wq0 + L
        wq1 = jnp.where(wq1 >= D, wq1 - D, wq1)
        return tp - wqv + pa, (fb * fr, fb * om, mdp * fr, mdp * om), wq1

    def gather4(gi):
        gn = gi + 1
        return (plsc.load_gather(h0, [gi]), plsc.load_gather(h0, [gn]),
                plsc.load_gather(h1, [gi]), plsc.load_gather(h1, [gn]))

    def emit_group(off, coef, taps):
        bfr, bom, mfr, mom = coef
        pv0, nv0, pv1, nv1 = taps
        for (xr, hr, orr, pv, nv) in ((x0, h0, o0, pv0, nv0),
                                      (x1, h1, o1, pv1, nv1)):
            xv = xr[pl.ds(pl.multiple_of(off, L), L)]
            hr[pl.ds(pl.multiple_of(off + PAD, L), L)] = \
                (xv + bom * pv) + bfr * nv
            orr[pl.ds(pl.multiple_of(off, L), L)] = \
                (xv + mom * pv) + mfr * nv

    g0, f0, wq1 = idxmath(jnp.int32(0), jnp.int32(0))
    g1, f1, wq2 = idxmath(jnp.int32(L), wq1)
    taps0 = gather4(g0)

    for cp in cps_b:
        cp.wait()

    def step(jj, carry):
        (fa0, fa1, fa2, fa3, ta0, ta1, ta2, ta3,
         gb, fb0, fb1, fb2, fb3, wqc) = carry
        off_a = jj * (2 * L)
        taps_b = gather4(gb)
        emit_group(off_a, (fa0, fa1, fa2, fa3), (ta0, ta1, ta2, ta3))
        gc, cfc, wqd = idxmath(off_a + 2 * L, wqc)
        taps_c = gather4(gc)
        emit_group(off_a + L, (fb0, fb1, fb2, fb3), taps_b)
        gd, cfd, wqe = idxmath(off_a + 3 * L, wqd)
        return (*cfc, *taps_c, gd, *cfd, wqe)

    half = lax.fori_loop(0, NG // 4, step, (*f0, *taps0, g1, *f1, wq2))
    ocp = [pltpu.async_copy(o0.at[pl.ds(0, N // 2)],
                            out_hbm.at[b, 0, pl.ds(0, N // 2)], sem_o),
           pltpu.async_copy(o1.at[pl.ds(0, N // 2)],
                            out_hbm.at[b, 1, pl.ds(0, N // 2)], sem_o)]
    lax.fori_loop(NG // 4, NG // 2, step, half)

    pltpu.sync_copy(o0.at[pl.ds(N // 2, N // 2)],
                    out_hbm.at[b, 0, pl.ds(N // 2, N // 2)])
    pltpu.sync_copy(o1.at[pl.ds(N // 2, N // 2)],
                    out_hbm.at[b, 1, pl.ds(N // 2, N // 2)])
    for cp in ocp:
        cp.wait()


@jax.jit
def kernel(x, mod_sig, feedback, width, depth, mix):
    mesh = plsc.VectorSubcoreMesh(core_axis_name="c", subcore_axis_name="s")
    f = functools.partial(
        pl.kernel,
        mesh=mesh,
        compiler_params=pltpu.CompilerParams(needs_layout_passes=False),
        out_type=jax.ShapeDtypeStruct((B, C, N), jnp.float32),
        scratch_types=[
            pltpu.VMEM((N + 2 * L,), jnp.float32),
            pltpu.VMEM((N,), jnp.float32),
            pltpu.VMEM((N,), jnp.float32),
            pltpu.VMEM((PAD + N,), jnp.float32),
            pltpu.VMEM((PAD + N,), jnp.float32),
            pltpu.VMEM((N,), jnp.float32),
            pltpu.VMEM((N,), jnp.float32),
            pltpu.VMEM((4 * B,), jnp.float32),
            pltpu.SemaphoreType.DMA,
            pltpu.SemaphoreType.DMA,
            pltpu.SemaphoreType.DMA,
        ],
    )(_flanger_body)
    return f(x, mod_sig, feedback, width, depth, mix)
